# baseline (speedup 1.0000x reference)
import functools

import jax
import jax.numpy as jnp
from jax import lax
from jax.experimental import pallas as pl
from jax.experimental.pallas import tpu as pltpu
from jax.experimental.pallas import tpu_sc as plsc

_EMBED = 32
_BATCH = 16384
_HIST = 50
_LINES = 250000
_BB = 128
_NBLK = _HIST * (_BATCH // _BB)
_NW = 32
_BLK_PER_W = _NBLK // _NW

_mesh = plsc.VectorSubcoreMesh(core_axis_name="c", subcore_axis_name="s")


@functools.partial(
    pl.kernel,
    mesh=_mesh,
    out_type=jax.ShapeDtypeStruct((_HIST, _EMBED, _BATCH), jnp.float32),
    scratch_types=[
        pltpu.VMEM((_BB,), jnp.int32),
        pltpu.VMEM((_BB,), jnp.int32),
        pltpu.VMEM((_BB,), jnp.int32),
        pltpu.VMEM((_BB,), jnp.int32),
        pltpu.VMEM((_BB,), jnp.int32),
        pltpu.VMEM((_BB,), jnp.int32),
        pltpu.VMEM((_BB, 128), jnp.float32),
        pltpu.VMEM((_BB, 128), jnp.float32),
        pltpu.VMEM((_EMBED, _BB), jnp.float32),
        pltpu.VMEM((_EMBED, _BB), jnp.float32),
        pltpu.SemaphoreType.DMA,
        pltpu.SemaphoreType.DMA,
        pltpu.SemaphoreType.DMA,
        pltpu.SemaphoreType.DMA,
    ],
    compiler_params=pltpu.CompilerParams(needs_layout_passes=False),
)
def _embed_gather(idxt_hbm, table_hbm, out_hbm,
                  idx_v0, idx_v1, row_v0, row_v1, sub_v0, sub_v1,
                  gath_v0, gath_v1, st_v0, st_v1,
                  gsem0, gsem1, ssem0, ssem1):
    wid = lax.axis_index("s") * 2 + lax.axis_index("c")
    base = wid * _BLK_PER_W

    def hb(c):
        blk = base + c
        return blk // (_BATCH // _BB), (blk % (_BATCH // _BB)) * _BB

    def load_and_fire(c, idx_v, row_v, sub_v, gath_v, gsem):
        h, b0 = hb(c)
        pltpu.sync_copy(idxt_hbm.at[h, pl.ds(b0, _BB)], idx_v)
        for jb in range(_BB // 16):
            v = idx_v[pl.ds(jb * 16, 16)]
            row_v[pl.ds(jb * 16, 16)] = lax.shift_right_logical(v, 2)
            sub_v[pl.ds(jb * 16, 16)] = lax.shift_left(
                lax.bitwise_and(v, 3), 5)
        pltpu.async_copy(table_hbm.at[row_v], gath_v, gsem)

    def wait_gather(gath_v, row_v, gsem):
        pltpu.make_async_copy(table_hbm.at[row_v], gath_v, gsem).wait()

    def select(sub_v, gath_v, st_v):
        iota16 = lax.iota(jnp.int32, 16)
        rows = [jb * 16 + iota16 for jb in range(_BB // 16)]
        subs = [sub_v[pl.ds(jb * 16, 16)] for jb in range(_BB // 16)]

        for e in range(_EMBED):
            for jb in range(_BB // 16):
                vals = plsc.load_gather(gath_v, [rows[jb], subs[jb] + e])
                st_v[e, pl.ds(jb * 16, 16)] = vals

    def fire_store(c, st_v, ssem):
        h, b0 = hb(c)
        pltpu.async_copy(st_v, out_hbm.at[h, :, pl.ds(b0, _BB)], ssem)

    def wait_store(c, st_v, ssem):
        h, b0 = hb(c)
        pltpu.make_async_copy(st_v, out_hbm.at[h, :, pl.ds(b0, _BB)],
                              ssem).wait()

    load_and_fire(0, idx_v0, row_v0, sub_v0, gath_v0, gsem0)

    def body(g, carry):
        c0 = 2 * g

        @pl.when(g > 0)
        def _():
            wait_store(c0 - 1, st_v1, ssem1)

        load_and_fire(c0 + 1, idx_v1, row_v1, sub_v1, gath_v1, gsem1)
        wait_gather(gath_v0, row_v0, gsem0)
        select(sub_v0, gath_v0, st_v0)
        fire_store(c0, st_v0, ssem0)

        wait_store(c0, st_v0, ssem0)

        @pl.when(g < _BLK_PER_W // 2 - 1)
        def _():
            load_and_fire(c0 + 2, idx_v0, row_v0, sub_v0, gath_v0, gsem0)

        wait_gather(gath_v1, row_v1, gsem1)
        select(sub_v1, gath_v1, st_v1)
        fire_store(c0 + 1, st_v1, ssem1)
        return carry

    lax.fori_loop(0, _BLK_PER_W // 2, body, 0)
    wait_store(_BLK_PER_W - 1, st_v1, ssem1)


def kernel(label_ids, table):
    idx_t = jnp.transpose(label_ids, (1, 0))
    table_lines = jnp.reshape(table, (_LINES, 128))
    out_t = _embed_gather(idx_t, table_lines)
    return jnp.transpose(out_t, (2, 0, 1))


# speedup vs baseline: 1.3385x; 1.3385x over previous
import functools

import jax
import jax.numpy as jnp
from jax import lax
from jax.experimental import pallas as pl
from jax.experimental.pallas import tpu as pltpu
from jax.experimental.pallas import tpu_sc as plsc

_EMBED = 32
_BATCH = 16384
_HIST = 50
_LINES = 250000
_BB = 128
_NBLK = _HIST * (_BATCH // _BB)
_NW = 32
_BLK_PER_W = _NBLK // _NW

_mesh = plsc.VectorSubcoreMesh(core_axis_name="c", subcore_axis_name="s")


@functools.partial(
    pl.kernel,
    mesh=_mesh,
    out_type=jax.ShapeDtypeStruct((_HIST, _EMBED, _BATCH), jnp.float32),
    scratch_types=[
        pltpu.VMEM((_BB,), jnp.int32),
        pltpu.VMEM((_BB,), jnp.int32),
        pltpu.VMEM((_BB,), jnp.int32),
        pltpu.VMEM((_BB,), jnp.int32),
        pltpu.VMEM((_BB,), jnp.int32),
        pltpu.VMEM((_BB,), jnp.int32),
        pltpu.VMEM((_BB, 128), jnp.float32),
        pltpu.VMEM((_BB, 128), jnp.float32),
        pltpu.VMEM((_EMBED, _BB), jnp.float32),
        pltpu.VMEM((_EMBED, _BB), jnp.float32),
        pltpu.SemaphoreType.DMA,
        pltpu.SemaphoreType.DMA,
        pltpu.SemaphoreType.DMA,
        pltpu.SemaphoreType.DMA,
    ],
    compiler_params=pltpu.CompilerParams(needs_layout_passes=False),
)
def _embed_gather(idxt_hbm, table_hbm, out_hbm,
                  idx_v0, idx_v1, row_v0, row_v1, sub_v0, sub_v1,
                  gath_v0, gath_v1, st_v0, st_v1,
                  gsem0, gsem1, ssem0, ssem1):
    wid = lax.axis_index("s") * 2 + lax.axis_index("c")
    base = wid * _BLK_PER_W

    def hb(c):
        blk = base + c
        return blk // (_BATCH // _BB), (blk % (_BATCH // _BB)) * _BB

    def load_and_fire(c, idx_v, row_v, sub_v, gath_v, gsem):
        h, b0 = hb(c)
        pltpu.sync_copy(idxt_hbm.at[h, pl.ds(b0, _BB)], idx_v)
        for jb in range(_BB // 16):
            v = idx_v[pl.ds(jb * 16, 16)]
            row_v[pl.ds(jb * 16, 16)] = lax.shift_right_logical(v, 2)
            sub_v[pl.ds(jb * 16, 16)] = lax.shift_left(
                lax.bitwise_and(v, 3), 5)
        pltpu.async_copy(table_hbm.at[row_v], gath_v, gsem)

    def wait_gather(gath_v, row_v, gsem):
        pltpu.make_async_copy(table_hbm.at[row_v], gath_v, gsem).wait()

    def select(sub_v, gath_v, st_v):
        iota16 = lax.iota(jnp.int32, 16)
        rows = [jb * 16 + iota16 for jb in range(_BB // 16)]
        subs = [sub_v[pl.ds(jb * 16, 16)] for jb in range(_BB // 16)]

        @plsc.parallel_loop(0, _EMBED, unroll=8)
        def e_body(e):
            for jb in range(_BB // 16):
                vals = plsc.load_gather(gath_v, [rows[jb], subs[jb] + e])
                st_v[e, pl.ds(jb * 16, 16)] = vals

    def fire_store(c, st_v, ssem):
        h, b0 = hb(c)
        pltpu.async_copy(st_v, out_hbm.at[h, :, pl.ds(b0, _BB)], ssem)

    def wait_store(c, st_v, ssem):
        h, b0 = hb(c)
        pltpu.make_async_copy(st_v, out_hbm.at[h, :, pl.ds(b0, _BB)],
                              ssem).wait()

    load_and_fire(0, idx_v0, row_v0, sub_v0, gath_v0, gsem0)

    def body(g, carry):
        c0 = 2 * g

        @pl.when(g > 0)
        def _():
            wait_store(c0 - 1, st_v1, ssem1)

        load_and_fire(c0 + 1, idx_v1, row_v1, sub_v1, gath_v1, gsem1)
        wait_gather(gath_v0, row_v0, gsem0)
        select(sub_v0, gath_v0, st_v0)
        fire_store(c0, st_v0, ssem0)

        wait_store(c0, st_v0, ssem0)

        @pl.when(g < _BLK_PER_W // 2 - 1)
        def _():
            load_and_fire(c0 + 2, idx_v0, row_v0, sub_v0, gath_v0, gsem0)

        wait_gather(gath_v1, row_v1, gsem1)
        select(sub_v1, gath_v1, st_v1)
        fire_store(c0 + 1, st_v1, ssem1)
        return carry

    lax.fori_loop(0, _BLK_PER_W // 2, body, 0)
    wait_store(_BLK_PER_W - 1, st_v1, ssem1)


def kernel(label_ids, table):
    idx_t = jnp.transpose(label_ids, (1, 0))
    table_lines = jnp.reshape(table, (_LINES, 128))
    out_t = _embed_gather(idx_t, table_lines)
    return jnp.transpose(out_t, (2, 0, 1))


# speedup vs baseline: 1.4146x; 1.0568x over previous
import functools

import jax
import jax.numpy as jnp
from jax import lax
from jax.experimental import pallas as pl
from jax.experimental.pallas import tpu as pltpu
from jax.experimental.pallas import tpu_sc as plsc

_EMBED = 32
_BATCH = 16384
_HIST = 50
_LINES = 250000
_BB = 256
_NBLK = _HIST * (_BATCH // _BB)
_NW = 32
_BLK_PER_W = _NBLK // _NW

_mesh = plsc.VectorSubcoreMesh(core_axis_name="c", subcore_axis_name="s")


@functools.partial(
    pl.kernel,
    mesh=_mesh,
    out_type=jax.ShapeDtypeStruct((_HIST, _EMBED, _BATCH), jnp.float32),
    scratch_types=[
        pltpu.VMEM((_BB,), jnp.int32),
        pltpu.VMEM((_BB,), jnp.int32),
        pltpu.VMEM((_BB,), jnp.int32),
        pltpu.VMEM((_BB,), jnp.int32),
        pltpu.VMEM((_BB,), jnp.int32),
        pltpu.VMEM((_BB,), jnp.int32),
        pltpu.VMEM((_BB, 128), jnp.float32),
        pltpu.VMEM((_BB, 128), jnp.float32),
        pltpu.VMEM((_EMBED, _BB), jnp.float32),
        pltpu.VMEM((_EMBED, _BB), jnp.float32),
        pltpu.SemaphoreType.DMA,
        pltpu.SemaphoreType.DMA,
        pltpu.SemaphoreType.DMA,
        pltpu.SemaphoreType.DMA,
    ],
    compiler_params=pltpu.CompilerParams(needs_layout_passes=False),
)
def _embed_gather(idxt_hbm, table_hbm, out_hbm,
                  idx_v0, idx_v1, row_v0, row_v1, sub_v0, sub_v1,
                  gath_v0, gath_v1, st_v0, st_v1,
                  gsem0, gsem1, ssem0, ssem1):
    wid = lax.axis_index("s") * 2 + lax.axis_index("c")
    base = wid * _BLK_PER_W

    def hb(c):
        blk = base + c
        return blk // (_BATCH // _BB), (blk % (_BATCH // _BB)) * _BB

    def load_and_fire(c, idx_v, row_v, sub_v, gath_v, gsem):
        h, b0 = hb(c)
        pltpu.sync_copy(idxt_hbm.at[h, pl.ds(b0, _BB)], idx_v)
        for jb in range(_BB // 16):
            v = idx_v[pl.ds(jb * 16, 16)]
            row_v[pl.ds(jb * 16, 16)] = lax.shift_right_logical(v, 2)
            sub_v[pl.ds(jb * 16, 16)] = lax.shift_left(
                lax.bitwise_and(v, 3), 5)
        pltpu.async_copy(table_hbm.at[row_v], gath_v, gsem)

    def wait_gather(gath_v, row_v, gsem):
        pltpu.make_async_copy(table_hbm.at[row_v], gath_v, gsem).wait()

    def select(sub_v, gath_v, st_v):
        iota16 = lax.iota(jnp.int32, 16)
        rows = [jb * 16 + iota16 for jb in range(_BB // 16)]
        subs = [sub_v[pl.ds(jb * 16, 16)] for jb in range(_BB // 16)]

        @plsc.parallel_loop(0, _EMBED, unroll=8)
        def e_body(e):
            for jb in range(_BB // 16):
                vals = plsc.load_gather(gath_v, [rows[jb], subs[jb] + e])
                st_v[e, pl.ds(jb * 16, 16)] = vals

    def fire_store(c, st_v, ssem):
        h, b0 = hb(c)
        pltpu.async_copy(st_v, out_hbm.at[h, :, pl.ds(b0, _BB)], ssem)

    def wait_store(c, st_v, ssem):
        h, b0 = hb(c)
        pltpu.make_async_copy(st_v, out_hbm.at[h, :, pl.ds(b0, _BB)],
                              ssem).wait()

    load_and_fire(0, idx_v0, row_v0, sub_v0, gath_v0, gsem0)

    def body(g, carry):
        c0 = 2 * g

        @pl.when(g > 0)
        def _():
            wait_store(c0 - 1, st_v1, ssem1)

        load_and_fire(c0 + 1, idx_v1, row_v1, sub_v1, gath_v1, gsem1)
        wait_gather(gath_v0, row_v0, gsem0)
        select(sub_v0, gath_v0, st_v0)
        fire_store(c0, st_v0, ssem0)

        wait_store(c0, st_v0, ssem0)

        @pl.when(g < _BLK_PER_W // 2 - 1)
        def _():
            load_and_fire(c0 + 2, idx_v0, row_v0, sub_v0, gath_v0, gsem0)

        wait_gather(gath_v1, row_v1, gsem1)
        select(sub_v1, gath_v1, st_v1)
        fire_store(c0 + 1, st_v1, ssem1)
        return carry

    lax.fori_loop(0, _BLK_PER_W // 2, body, 0)
    wait_store(_BLK_PER_W - 1, st_v1, ssem1)


def kernel(label_ids, table):
    idx_t = jnp.transpose(label_ids, (1, 0))
    table_lines = jnp.reshape(table, (_LINES, 128))
    out_t = _embed_gather(idx_t, table_lines)
    return jnp.transpose(out_t, (2, 0, 1))
